# baseline (speedup 1.0000x reference)
import functools

import jax
import jax.numpy as jnp
from jax import lax
from jax.experimental import pallas as pl
from jax.experimental.pallas import tpu as pltpu
from jax.experimental.pallas import tpu_sc as plsc

NC = 2
NS = 16
NW = NC * NS
K = 128
G = 64
PAD = 8
ALPHA = 0.84


def _sc_mesh():
    return plsc.VectorSubcoreMesh(
        core_axis_name="c", subcore_axis_name="s", num_cores=NC,
        num_subcores=NS)


def _tile_slab_copy(src, dst, sid, N):
    rpt = (N // NS) // 8 * 8
    tail = N - NS * rpt
    pltpu.sync_copy(src.at[pl.ds(sid * rpt, rpt)],
                    dst.at[pl.ds(sid * rpt, rpt)])
    if tail:
        @pl.when(sid == NS - 1)
        def _():
            pltpu.sync_copy(src.at[pl.ds(NS * rpt, tail)],
                            dst.at[pl.ds(NS * rpt, tail)])


def _make_deg_kernel(N, H, C0, C1):
    @functools.partial(
        pl.kernel,
        out_type=jax.ShapeDtypeStruct((NC, N, H), jnp.float32),
        mesh=_sc_mesh(),
        scratch_types=[
            pltpu.VMEM((1, K), jnp.int32),
            pltpu.VMEM((1, K), jnp.int32),
            pltpu.VMEM((K, H), jnp.float32),
            pltpu.VMEM_SHARED((N + PAD, H), jnp.float32),
            pltpu.SemaphoreType.DMA,
            pltpu.SemaphoreType.DMA,
        ],
    )
    def deg_kernel(dst_hbm, ones_hbm, zeros_hbm, out_hbm,
                   i0, i1, onesv, acc, semi0, semi1):
        cid = lax.axis_index("c")
        sid = lax.axis_index("s")
        wid = cid * NS + sid
        cw = jnp.where(cid == 0, C0, C1)
        my_dst = dst_hbm.at[wid]
        pltpu.sync_copy(ones_hbm, onesv)
        _tile_slab_copy(zeros_hbm, acc, sid, N + PAD)
        plsc.subcore_barrier()

        pltpu.async_copy(my_dst.at[0], i0, semi0)

        @pl.loop(0, cw // 2)
        def _(p):
            i = 2 * p
            pltpu.make_async_copy(my_dst.at[i], i0, semi0).wait()
            pltpu.async_copy(my_dst.at[i + 1], i1, semi1)
            pltpu.sync_copy(onesv, acc.at[i0.at[0]], add=True)
            pltpu.make_async_copy(my_dst.at[i + 1], i1, semi1).wait()

            @pl.when(i + 2 < cw)
            def _():
                pltpu.async_copy(my_dst.at[i + 2], i0, semi0)

            pltpu.sync_copy(onesv, acc.at[i1.at[0]], add=True)

        plsc.subcore_barrier()
        _tile_slab_copy(acc, out_hbm.at[cid], sid, N)

    return deg_kernel


def _make_agg_kernel(N, H, C0, C1):
    @functools.partial(
        pl.kernel,
        out_type=jax.ShapeDtypeStruct((NC, N, H), jnp.float32),
        mesh=_sc_mesh(),
        scratch_types=(
            [pltpu.VMEM((1, K), jnp.int32)] * 4 +
            [pltpu.VMEM((K, H), jnp.float32)] * 2 +
            [pltpu.VMEM_SHARED((N + PAD, H), jnp.float32)] +
            [pltpu.SemaphoreType.DMA] * 6
        ),
    )
    def agg_kernel(g_hbm, src_hbm, dst_hbm, zeros_hbm, out_hbm,
                   s0, s1, d0, d1, bufa, bufb, acc, *sems):
        ss0, ss1, ds0, ds1, ga, gb = sems
        cid = lax.axis_index("c")
        sid = lax.axis_index("s")
        wid = cid * NS + sid
        cw = jnp.where(cid == 0, C0, C1)
        my_src = src_hbm.at[wid]
        my_dst = dst_hbm.at[wid]
        _tile_slab_copy(zeros_hbm, acc, sid, N + PAD)
        plsc.subcore_barrier()

        pltpu.async_copy(my_src.at[0], s0, ss0)
        pltpu.async_copy(my_src.at[1], s1, ss1)
        pltpu.async_copy(my_dst.at[0], d0, ds0)
        pltpu.async_copy(my_dst.at[1], d1, ds1)
        pltpu.make_async_copy(my_src.at[0], s0, ss0).wait()
        pltpu.async_copy(g_hbm.at[s0.at[0]], bufa, ga)

        @pl.loop(0, cw // 2)
        def _(pr):
            i = 2 * pr
            pltpu.make_async_copy(my_src.at[i + 1], s1, ss1).wait()
            pltpu.async_copy(g_hbm.at[s1.at[0]], bufb, gb)
            pltpu.make_async_copy(g_hbm.at[s0.at[0]], bufa, ga).wait()

            @pl.when(i + 2 < cw)
            def _():
                pltpu.async_copy(my_src.at[i + 2], s0, ss0)

            pltpu.make_async_copy(my_dst.at[i], d0, ds0).wait()
            pltpu.sync_copy(bufa, acc.at[d0.at[0]], add=True)

            @pl.when(i + 2 < cw)
            def _():
                pltpu.async_copy(my_dst.at[i + 2], d0, ds0)
                pltpu.make_async_copy(my_src.at[i + 2], s0, ss0).wait()
                pltpu.async_copy(g_hbm.at[s0.at[0]], bufa, ga)

            pltpu.make_async_copy(g_hbm.at[s1.at[0]], bufb, gb).wait()

            @pl.when(i + 3 < cw)
            def _():
                pltpu.async_copy(my_src.at[i + 3], s1, ss1)

            pltpu.make_async_copy(my_dst.at[i + 1], d1, ds1).wait()
            pltpu.sync_copy(bufb, acc.at[d1.at[0]], add=True)

            @pl.when(i + 3 < cw)
            def _():
                pltpu.async_copy(my_dst.at[i + 3], d1, ds1)

        plsc.subcore_barrier()
        _tile_slab_copy(acc, out_hbm.at[cid], sid, N)

    return agg_kernel


def _prep_body(degp_ref, x_ref, w_ref, dinv_ref, g_ref):
    deg = degp_ref[0, :, 0:1] + degp_ref[1, :, 0:1] + 1.0
    dinv = lax.rsqrt(deg)
    dinv_ref[...] = dinv
    xw = jnp.dot(x_ref[...], w_ref[...], preferred_element_type=jnp.float32)
    g_ref[...] = xw * dinv


def _combine_body(agg_ref, g_ref, dinv_ref, b_ref, w_ref,
                  h_ref, r_ref, gn_ref):
    dinv = dinv_ref[...]
    s = agg_ref[0] + agg_ref[1] + g_ref[...]
    h = dinv * s + b_ref[...]
    h_ref[...] = h
    r = jnp.maximum(h, 0.0)
    r_ref[...] = r
    gn_ref[...] = dinv * jnp.dot(r, w_ref[...],
                                 preferred_element_type=jnp.float32)


def _make_combine3_body(nb):
    def body(agg_ref, g_ref, dinv_ref, b_ref, batch_ref,
             h4_ref, h5_ref, sm_ref, sums_acc, cnt_acc):
        i = pl.program_id(0)
        s = agg_ref[0] + agg_ref[1] + g_ref[...]
        h4 = dinv_ref[...] * s + b_ref[...]
        h4_ref[...] = h4

        bb = batch_ref[0]
        gid = lax.broadcasted_iota(jnp.int32, (G, bb.shape[1]), 0)
        p = (gid == bb).astype(jnp.float32)

        @pl.when(i == 0)
        def _():
            sums_acc[...] = jnp.zeros_like(sums_acc)
            cnt_acc[...] = jnp.zeros_like(cnt_acc)

        sums_acc[...] += jnp.dot(p, h4, preferred_element_type=jnp.float32)
        cnt_acc[...] += jnp.sum(p, axis=1, keepdims=True)

        @pl.when(i == nb - 1)
        def _():
            h5 = sums_acc[...] / jnp.maximum(cnt_acc[...], 1.0)
            h5_ref[...] = h5
            m = jnp.max(h5, axis=1, keepdims=True)
            e = jnp.exp(h5 - m)
            sm_ref[...] = e / jnp.sum(e, axis=1, keepdims=True)

    return body


def kernel(x, edge_index, batch, W1, b1, W2, b2, W3, b3):
    N, D = x.shape
    H = W1.shape[1]
    E = edge_index.shape[1]
    ct = -(-E // (NS * K))
    C0 = max(2, -(-int(ct * ALPHA) // 2) * 2)
    C1 = max(2, -(-(ct - C0) // 2) * 2)
    Ep = NS * K * (C0 + C1)
    pad = jnp.full((2, Ep - E), N, jnp.int32)
    eidx = jnp.concatenate([edge_index, pad], axis=1)
    cmax = max(C0, C1)
    a = eidx[:, :NS * C0 * K].reshape(2, NS, C0, 1, K)
    b = eidx[:, NS * C0 * K:].reshape(2, NS, C1, 1, K)
    if C1 < cmax:
        bpad = jnp.full((2, NS, cmax - C1, 1, K), N, jnp.int32)
        b = jnp.concatenate([b, bpad], axis=2)
    elif C0 < cmax:
        apad = jnp.full((2, NS, cmax - C0, 1, K), N, jnp.int32)
        a = jnp.concatenate([a, apad], axis=2)
    full = jnp.concatenate([a, b], axis=1)
    src_r = full[0]
    dst_r = full[1]
    zeros_nh = jnp.zeros((N + PAD, H), jnp.float32)
    ones_kh = jnp.ones((K, H), jnp.float32)

    deg_kernel = _make_deg_kernel(N, H, C0, C1)
    agg_kernel = _make_agg_kernel(N, H, C0, C1)

    degp = deg_kernel(dst_r, ones_kh, zeros_nh)

    B = 1000
    nb = N // B
    grid = (nb,)
    row_b = lambda i: (i, 0)
    full_b = lambda i: (0, 0)

    b1r = b1.reshape(1, H)
    b2r = b2.reshape(1, H)
    b3r = b3.reshape(1, H)
    batch_r = batch.reshape(nb, 1, B)

    dinv, g1 = pl.pallas_call(
        _prep_body,
        grid=grid,
        in_specs=[
            pl.BlockSpec((NC, B, H), lambda i: (0, i, 0)),
            pl.BlockSpec((B, D), row_b),
            pl.BlockSpec((D, H), full_b),
        ],
        out_specs=[
            pl.BlockSpec((B, 1), row_b),
            pl.BlockSpec((B, H), row_b),
        ],
        out_shape=[
            jax.ShapeDtypeStruct((N, 1), jnp.float32),
            jax.ShapeDtypeStruct((N + PAD, H), jnp.float32),
        ],
    )(degp, x, W1)

    def combine(agg, g, bias, w_next):
        return pl.pallas_call(
            _combine_body,
            grid=grid,
            in_specs=[
                pl.BlockSpec((NC, B, H), lambda i: (0, i, 0)),
                pl.BlockSpec((B, H), row_b),
                pl.BlockSpec((B, 1), row_b),
                pl.BlockSpec((1, H), full_b),
                pl.BlockSpec((H, H), full_b),
            ],
            out_specs=[
                pl.BlockSpec((B, H), row_b),
                pl.BlockSpec((B, H), row_b),
                pl.BlockSpec((B, H), row_b),
            ],
            out_shape=[
                jax.ShapeDtypeStruct((N, H), jnp.float32),
                jax.ShapeDtypeStruct((N, H), jnp.float32),
                jax.ShapeDtypeStruct((N + PAD, H), jnp.float32),
            ],
        )(agg, g, dinv, bias, w_next)

    agg1 = agg_kernel(g1, src_r, dst_r, zeros_nh)
    h, h1, g2 = combine(agg1, g1, b1r, W2)
    agg2 = agg_kernel(g2, src_r, dst_r, zeros_nh)
    h2, h3, g3 = combine(agg2, g2, b2r, W3)
    agg3 = agg_kernel(g3, src_r, dst_r, zeros_nh)

    h4, h5, out = pl.pallas_call(
        _make_combine3_body(nb),
        grid=grid,
        in_specs=[
            pl.BlockSpec((NC, B, H), lambda i: (0, i, 0)),
            pl.BlockSpec((B, H), row_b),
            pl.BlockSpec((B, 1), row_b),
            pl.BlockSpec((1, H), full_b),
            pl.BlockSpec((1, 1, B), lambda i: (i, 0, 0)),
        ],
        out_specs=[
            pl.BlockSpec((B, H), row_b),
            pl.BlockSpec((G, H), full_b),
            pl.BlockSpec((G, H), full_b),
        ],
        out_shape=[
            jax.ShapeDtypeStruct((N, H), jnp.float32),
            jax.ShapeDtypeStruct((G, H), jnp.float32),
            jax.ShapeDtypeStruct((G, H), jnp.float32),
        ],
        scratch_shapes=[
            pltpu.VMEM((G, H), jnp.float32),
            pltpu.VMEM((G, 1), jnp.float32),
        ],
    )(agg3, g3, dinv, b3r, batch_r)

    return (out, h, h1, h2, h3, h4, h5, h5)


# speedup vs baseline: 1.0256x; 1.0256x over previous
import functools

import jax
import jax.numpy as jnp
from jax import lax
from jax.experimental import pallas as pl
from jax.experimental.pallas import tpu as pltpu
from jax.experimental.pallas import tpu_sc as plsc

NC = 2
NS = 16
NW = NC * NS
K = 128
G = 64
PAD = 8
ALPHA = 0.90


def _sc_mesh():
    return plsc.VectorSubcoreMesh(
        core_axis_name="c", subcore_axis_name="s", num_cores=NC,
        num_subcores=NS)


def _tile_slab_copy(src, dst, sid, N):
    rpt = (N // NS) // 8 * 8
    tail = N - NS * rpt
    pltpu.sync_copy(src.at[pl.ds(sid * rpt, rpt)],
                    dst.at[pl.ds(sid * rpt, rpt)])
    if tail:
        @pl.when(sid == NS - 1)
        def _():
            pltpu.sync_copy(src.at[pl.ds(NS * rpt, tail)],
                            dst.at[pl.ds(NS * rpt, tail)])


def _make_deg_kernel(N, H, C0, C1):
    @functools.partial(
        pl.kernel,
        out_type=jax.ShapeDtypeStruct((NC, N, H), jnp.float32),
        mesh=_sc_mesh(),
        scratch_types=[
            pltpu.VMEM((1, K), jnp.int32),
            pltpu.VMEM((1, K), jnp.int32),
            pltpu.VMEM((K, H), jnp.float32),
            pltpu.VMEM_SHARED((N + PAD, H), jnp.float32),
            pltpu.SemaphoreType.DMA,
            pltpu.SemaphoreType.DMA,
        ],
    )
    def deg_kernel(dst_hbm, ones_hbm, zeros_hbm, out_hbm,
                   i0, i1, onesv, acc, semi0, semi1):
        cid = lax.axis_index("c")
        sid = lax.axis_index("s")
        wid = cid * NS + sid
        cw = jnp.where(cid == 0, C0, C1)
        my_dst = dst_hbm.at[wid]
        pltpu.sync_copy(ones_hbm, onesv)
        _tile_slab_copy(zeros_hbm, acc, sid, N + PAD)
        plsc.subcore_barrier()

        pltpu.async_copy(my_dst.at[0], i0, semi0)

        @pl.loop(0, cw // 2)
        def _(p):
            i = 2 * p
            pltpu.make_async_copy(my_dst.at[i], i0, semi0).wait()
            pltpu.async_copy(my_dst.at[i + 1], i1, semi1)
            pltpu.sync_copy(onesv, acc.at[i0.at[0]], add=True)
            pltpu.make_async_copy(my_dst.at[i + 1], i1, semi1).wait()

            @pl.when(i + 2 < cw)
            def _():
                pltpu.async_copy(my_dst.at[i + 2], i0, semi0)

            pltpu.sync_copy(onesv, acc.at[i1.at[0]], add=True)

        plsc.subcore_barrier()
        _tile_slab_copy(acc, out_hbm.at[cid], sid, N)

    return deg_kernel


def _make_agg_kernel(N, H, C0, C1):
    @functools.partial(
        pl.kernel,
        out_type=jax.ShapeDtypeStruct((NC, N, H), jnp.float32),
        mesh=_sc_mesh(),
        scratch_types=(
            [pltpu.VMEM((1, K), jnp.int32)] * 4 +
            [pltpu.VMEM((K, H), jnp.float32)] * 2 +
            [pltpu.VMEM_SHARED((N + PAD, H), jnp.float32)] +
            [pltpu.SemaphoreType.DMA] * 6
        ),
    )
    def agg_kernel(g_hbm, src_hbm, dst_hbm, zeros_hbm, out_hbm,
                   s0, s1, d0, d1, bufa, bufb, acc, *sems):
        ss0, ss1, ds0, ds1, ga, gb = sems
        cid = lax.axis_index("c")
        sid = lax.axis_index("s")
        wid = cid * NS + sid
        cw = jnp.where(cid == 0, C0, C1)
        my_src = src_hbm.at[wid]
        my_dst = dst_hbm.at[wid]
        _tile_slab_copy(zeros_hbm, acc, sid, N + PAD)
        plsc.subcore_barrier()

        pltpu.async_copy(my_src.at[0], s0, ss0)
        pltpu.async_copy(my_src.at[1], s1, ss1)
        pltpu.async_copy(my_dst.at[0], d0, ds0)
        pltpu.async_copy(my_dst.at[1], d1, ds1)
        pltpu.make_async_copy(my_src.at[0], s0, ss0).wait()
        pltpu.async_copy(g_hbm.at[s0.at[0]], bufa, ga)

        @pl.loop(0, cw // 2)
        def _(pr):
            i = 2 * pr
            pltpu.make_async_copy(my_src.at[i + 1], s1, ss1).wait()
            pltpu.async_copy(g_hbm.at[s1.at[0]], bufb, gb)
            pltpu.make_async_copy(g_hbm.at[s0.at[0]], bufa, ga).wait()

            @pl.when(i + 2 < cw)
            def _():
                pltpu.async_copy(my_src.at[i + 2], s0, ss0)

            pltpu.make_async_copy(my_dst.at[i], d0, ds0).wait()
            pltpu.sync_copy(bufa, acc.at[d0.at[0]], add=True)

            @pl.when(i + 2 < cw)
            def _():
                pltpu.async_copy(my_dst.at[i + 2], d0, ds0)
                pltpu.make_async_copy(my_src.at[i + 2], s0, ss0).wait()
                pltpu.async_copy(g_hbm.at[s0.at[0]], bufa, ga)

            pltpu.make_async_copy(g_hbm.at[s1.at[0]], bufb, gb).wait()

            @pl.when(i + 3 < cw)
            def _():
                pltpu.async_copy(my_src.at[i + 3], s1, ss1)

            pltpu.make_async_copy(my_dst.at[i + 1], d1, ds1).wait()
            pltpu.sync_copy(bufb, acc.at[d1.at[0]], add=True)

            @pl.when(i + 3 < cw)
            def _():
                pltpu.async_copy(my_dst.at[i + 3], d1, ds1)

        plsc.subcore_barrier()
        _tile_slab_copy(acc, out_hbm.at[cid], sid, N)

    return agg_kernel


def _prep_body(degp_ref, x_ref, w_ref, dinv_ref, g_ref):
    deg = degp_ref[0, :, 0:1] + degp_ref[1, :, 0:1] + 1.0
    dinv = lax.rsqrt(deg)
    dinv_ref[...] = dinv
    xw = jnp.dot(x_ref[...], w_ref[...], preferred_element_type=jnp.float32)
    g_ref[...] = xw * dinv


def _combine_body(agg_ref, g_ref, dinv_ref, b_ref, w_ref,
                  h_ref, r_ref, gn_ref):
    dinv = dinv_ref[...]
    s = agg_ref[0] + agg_ref[1] + g_ref[...]
    h = dinv * s + b_ref[...]
    h_ref[...] = h
    r = jnp.maximum(h, 0.0)
    r_ref[...] = r
    gn_ref[...] = dinv * jnp.dot(r, w_ref[...],
                                 preferred_element_type=jnp.float32)


def _make_combine3_body(nb):
    def body(agg_ref, g_ref, dinv_ref, b_ref, batch_ref,
             h4_ref, h5_ref, sm_ref, sums_acc, cnt_acc):
        i = pl.program_id(0)
        s = agg_ref[0] + agg_ref[1] + g_ref[...]
        h4 = dinv_ref[...] * s + b_ref[...]
        h4_ref[...] = h4

        bb = batch_ref[0]
        gid = lax.broadcasted_iota(jnp.int32, (G, bb.shape[1]), 0)
        p = (gid == bb).astype(jnp.float32)

        @pl.when(i == 0)
        def _():
            sums_acc[...] = jnp.zeros_like(sums_acc)
            cnt_acc[...] = jnp.zeros_like(cnt_acc)

        sums_acc[...] += jnp.dot(p, h4, preferred_element_type=jnp.float32)
        cnt_acc[...] += jnp.sum(p, axis=1, keepdims=True)

        @pl.when(i == nb - 1)
        def _():
            h5 = sums_acc[...] / jnp.maximum(cnt_acc[...], 1.0)
            h5_ref[...] = h5
            m = jnp.max(h5, axis=1, keepdims=True)
            e = jnp.exp(h5 - m)
            sm_ref[...] = e / jnp.sum(e, axis=1, keepdims=True)

    return body


def kernel(x, edge_index, batch, W1, b1, W2, b2, W3, b3):
    N, D = x.shape
    H = W1.shape[1]
    E = edge_index.shape[1]
    ct = -(-E // (NS * K))
    C0 = max(2, -(-int(ct * ALPHA) // 2) * 2)
    C1 = max(2, -(-(ct - C0) // 2) * 2)
    Ep = NS * K * (C0 + C1)
    pad = jnp.full((2, Ep - E), N, jnp.int32)
    eidx = jnp.concatenate([edge_index, pad], axis=1)
    cmax = max(C0, C1)
    a = eidx[:, :NS * C0 * K].reshape(2, NS, C0, 1, K)
    b = eidx[:, NS * C0 * K:].reshape(2, NS, C1, 1, K)
    if C1 < cmax:
        bpad = jnp.full((2, NS, cmax - C1, 1, K), N, jnp.int32)
        b = jnp.concatenate([b, bpad], axis=2)
    elif C0 < cmax:
        apad = jnp.full((2, NS, cmax - C0, 1, K), N, jnp.int32)
        a = jnp.concatenate([a, apad], axis=2)
    full = jnp.concatenate([a, b], axis=1)
    src_r = full[0]
    dst_r = full[1]
    zeros_nh = jnp.zeros((N + PAD, H), jnp.float32)
    ones_kh = jnp.ones((K, H), jnp.float32)

    deg_kernel = _make_deg_kernel(N, H, C0, C1)
    agg_kernel = _make_agg_kernel(N, H, C0, C1)

    degp = deg_kernel(dst_r, ones_kh, zeros_nh)

    B = 1000
    nb = N // B
    grid = (nb,)
    row_b = lambda i: (i, 0)
    full_b = lambda i: (0, 0)

    b1r = b1.reshape(1, H)
    b2r = b2.reshape(1, H)
    b3r = b3.reshape(1, H)
    batch_r = batch.reshape(nb, 1, B)

    dinv, g1 = pl.pallas_call(
        _prep_body,
        grid=grid,
        in_specs=[
            pl.BlockSpec((NC, B, H), lambda i: (0, i, 0)),
            pl.BlockSpec((B, D), row_b),
            pl.BlockSpec((D, H), full_b),
        ],
        out_specs=[
            pl.BlockSpec((B, 1), row_b),
            pl.BlockSpec((B, H), row_b),
        ],
        out_shape=[
            jax.ShapeDtypeStruct((N, 1), jnp.float32),
            jax.ShapeDtypeStruct((N + PAD, H), jnp.float32),
        ],
    )(degp, x, W1)

    def combine(agg, g, bias, w_next):
        return pl.pallas_call(
            _combine_body,
            grid=grid,
            in_specs=[
                pl.BlockSpec((NC, B, H), lambda i: (0, i, 0)),
                pl.BlockSpec((B, H), row_b),
                pl.BlockSpec((B, 1), row_b),
                pl.BlockSpec((1, H), full_b),
                pl.BlockSpec((H, H), full_b),
            ],
            out_specs=[
                pl.BlockSpec((B, H), row_b),
                pl.BlockSpec((B, H), row_b),
                pl.BlockSpec((B, H), row_b),
            ],
            out_shape=[
                jax.ShapeDtypeStruct((N, H), jnp.float32),
                jax.ShapeDtypeStruct((N, H), jnp.float32),
                jax.ShapeDtypeStruct((N + PAD, H), jnp.float32),
            ],
        )(agg, g, dinv, bias, w_next)

    agg1 = agg_kernel(g1, src_r, dst_r, zeros_nh)
    h, h1, g2 = combine(agg1, g1, b1r, W2)
    agg2 = agg_kernel(g2, src_r, dst_r, zeros_nh)
    h2, h3, g3 = combine(agg2, g2, b2r, W3)
    agg3 = agg_kernel(g3, src_r, dst_r, zeros_nh)

    h4, h5, out = pl.pallas_call(
        _make_combine3_body(nb),
        grid=grid,
        in_specs=[
            pl.BlockSpec((NC, B, H), lambda i: (0, i, 0)),
            pl.BlockSpec((B, H), row_b),
            pl.BlockSpec((B, 1), row_b),
            pl.BlockSpec((1, H), full_b),
            pl.BlockSpec((1, 1, B), lambda i: (i, 0, 0)),
        ],
        out_specs=[
            pl.BlockSpec((B, H), row_b),
            pl.BlockSpec((G, H), full_b),
            pl.BlockSpec((G, H), full_b),
        ],
        out_shape=[
            jax.ShapeDtypeStruct((N, H), jnp.float32),
            jax.ShapeDtypeStruct((G, H), jnp.float32),
            jax.ShapeDtypeStruct((G, H), jnp.float32),
        ],
        scratch_shapes=[
            pltpu.VMEM((G, H), jnp.float32),
            pltpu.VMEM((G, 1), jnp.float32),
        ],
    )(agg3, g3, dinv, b3r, batch_r)

    return (out, h, h1, h2, h3, h4, h5, h5)
